# baseline (speedup 1.0000x reference)
import functools

import jax
import jax.numpy as jnp
from jax import lax
from jax.experimental import pallas as pl
from jax.experimental.pallas import tpu as pltpu
from jax.experimental.pallas import tpu_sc as plsc

S = 7
C = 20
BATCH = 4096
F = C + 10
CELLS = S * S
NFEAT = CELLS * F
OSTR = 336
PCELLS = 56

NC = 2
NS = 16
NW = NC * NS
IMGS_PER_W = BATCH // NW
CH = 16
N_CH = IMGS_PER_W // CH

GRP = 4
ROWW = GRP * NFEAT
NROWS = BATCH // GRP
ROWS_PER_CH = CH // GRP

_mesh = plsc.VectorSubcoreMesh(core_axis_name="c", subcore_axis_name="s")


def _make_sc_kernel(thresh):
    @functools.partial(
        pl.kernel,
        mesh=_mesh,
        compiler_params=pltpu.CompilerParams(needs_layout_passes=False),
        out_type=(
            jax.ShapeDtypeStruct((BATCH, OSTR), jnp.float32),
            jax.ShapeDtypeStruct((NW, IMGS_PER_W), jnp.int32),
        ),
        scratch_types=[
            pltpu.VMEM((ROWS_PER_CH, ROWW), jnp.float32),
            pltpu.VMEM((ROWS_PER_CH, ROWW), jnp.float32),
            pltpu.VMEM((CH, OSTR), jnp.float32),
            pltpu.VMEM((CH, OSTR), jnp.float32),
            pltpu.VMEM((IMGS_PER_W,), jnp.int32),
            pltpu.SemaphoreType.DMA,
            pltpu.SemaphoreType.DMA,
        ],
    )
    def _sc_map_kernel(src, dst, cnt_hbm, in0, in1, out0, out1, cnt_v,
                       in_sem, out_sem):
        wid = lax.axis_index("s") * NC + lax.axis_index("c")
        lane = lax.iota(jnp.int32, 16)
        inv_s = jnp.float32(1.0 / S)
        one = jnp.ones((16,), jnp.int32)
        zero_i = jnp.zeros((16,), jnp.int32)
        zero_f = jnp.zeros((16,), jnp.float32)

        cgs = []
        for cg in range(3):
            cellv = cg * 16 + lane
            cgs.append((cellv * F, cellv * 6,
                        (cellv % S).astype(jnp.float32),
                        (cellv // S).astype(jnp.float32)))

        def convert(gf, colf, rowf):
            pairs = [(gf(k), jnp.full((16,), jnp.float32(k)))
                     for k in range(C)]
            while len(pairs) > 1:
                nxt = []
                for j in range(0, len(pairs) - 1, 2):
                    (m1, l1), (m2, l2) = pairs[j], pairs[j + 1]
                    right = m2 > m1
                    nxt.append((jnp.where(right, m2, m1),
                                jnp.where(right, l2, l1)))
                if len(pairs) % 2:
                    nxt.append(pairs[-1])
                pairs = nxt
            label = pairs[0][1]

            conf1 = gf(C)
            conf2 = gf(C + 5)
            best = conf2 > conf1
            bb0 = jnp.where(best, gf(C + 6), gf(C + 1))
            bb1 = jnp.where(best, gf(C + 7), gf(C + 2))
            bb2 = jnp.where(best, gf(C + 8), gf(C + 3))
            bb3 = jnp.where(best, gf(C + 9), gf(C + 4))

            cx = (bb0 + colf) * inv_s
            cy = (bb1 + rowf) * inv_s
            w2 = bb2 * inv_s * 0.5
            h2 = bb3 * inv_s * 0.5
            conf = jnp.maximum(conf1, conf2)
            mask = conf > thresh
            outs = (cx - w2, cy - h2, cx + w2, cy + h2, conf, label)
            return mask, [jnp.where(mask, o, zero_f) for o in outs]

        def process_chunk(ci, in_ref, out_ref):
            def img_body(img, cnt_acc):
                imgv = jnp.full((16,), img, jnp.int32)
                rowv = jnp.full((16,), img // GRP, jnp.int32)
                coff = (img % GRP) * NFEAT
                msum = zero_i
                for base_in, base_out, colf, rowf in cgs:
                    def gf(f, _b=base_in):
                        return plsc.load_gather(in_ref, [rowv, coff + _b + f])

                    mask, outs = convert(gf, colf, rowf)
                    for k in range(6):
                        plsc.store_scatter(out_ref, [imgv, base_out + k],
                                           outs[k])
                    msum = msum + jnp.where(mask, one, zero_i)
                s = jnp.sum(msum)
                return jnp.where(lane == imgv,
                                 jnp.full((16,), s, jnp.int32), cnt_acc)

            cnt_acc = lax.fori_loop(0, CH, img_body, zero_i)

            c48 = (CELLS - 1) * F
            lrow = lane // GRP
            lcol = (lane % GRP) * NFEAT + c48

            def gf48(f):
                return plsc.load_gather(in_ref, [lrow, lcol + f])

            col48 = jnp.full((16,), jnp.float32((CELLS - 1) % S))
            row48 = jnp.full((16,), jnp.float32((CELLS - 1) // S))
            mask48, outs48 = convert(gf48, col48, row48)
            ob48 = (CELLS - 1) * 6
            for k in range(6):
                plsc.store_scatter(
                    out_ref, [lane, jnp.full((16,), ob48 + k, jnp.int32)],
                    outs48[k])
            cnt_acc = cnt_acc + jnp.where(mask48, one, zero_i)
            cnt_v[pl.ds(ci * CH, CH)] = cnt_acc

        base_img = wid * IMGS_PER_W
        bufs = ((in0, out0), (in1, out1))

        base_row = wid * (IMGS_PER_W // GRP)

        def start_in(ci, b):
            pltpu.async_copy(
                src.at[pl.ds(base_row + ci * ROWS_PER_CH, ROWS_PER_CH), :],
                bufs[b][0], in_sem)

        def wait_in(b):
            pltpu.make_async_copy(src.at[pl.ds(0, ROWS_PER_CH), :],
                                  bufs[b][0], in_sem).wait()

        def start_out(ci, b):
            pltpu.async_copy(bufs[b][1],
                             dst.at[pl.ds(base_img + ci * CH, CH), :],
                             out_sem)

        def wait_out(b):
            pltpu.make_async_copy(bufs[b][1], dst.at[pl.ds(0, CH), :],
                                  out_sem).wait()

        def half(ci, b, first_pair):
            wait_in(b)

            @pl.when(jnp.logical_not(first_pair))
            def _():
                wait_out(b)

            process_chunk(ci, bufs[b][0], bufs[b][1])
            start_out(ci, b)

            @pl.when(ci + 2 < N_CH)
            def _():
                start_in(ci + 2, b)

        start_in(0, 0)
        start_in(1, 1)

        def pair_body(cp, _):
            ci0 = cp * 2
            first = cp == 0
            half(ci0, 0, first)
            half(ci0 + 1, 1, first)
            return 0

        lax.fori_loop(0, N_CH // 2, pair_body, 0)
        wait_out(0)
        wait_out(1)
        pltpu.sync_copy(cnt_v, cnt_hbm.at[wid])

    return _sc_map_kernel


_pred_kernel = _make_sc_kernel(jnp.float32(0.1))
_tgt_kernel = _make_sc_kernel(jnp.float32(0.5))


def kernel(predictions, targets):
    mp, pc = _pred_kernel(predictions.reshape(NROWS, ROWW))
    mt, tc = _tgt_kernel(targets.reshape(NROWS, ROWW))
    return (mp.reshape(BATCH, PCELLS, 6)[:, :CELLS, :],
            mt.reshape(BATCH, PCELLS, 6)[:, :CELLS, :],
            pc.reshape(BATCH),
            tc.reshape(BATCH))


# speedup vs baseline: 1.2944x; 1.2944x over previous
import functools

import jax
import jax.numpy as jnp
from jax import lax
from jax.experimental import pallas as pl
from jax.experimental.pallas import tpu as pltpu
from jax.experimental.pallas import tpu_sc as plsc

S = 7
C = 20
BATCH = 4096
F = C + 10
CELLS = S * S
NFEAT = CELLS * F
OSTR = 336
PCELLS = 56

NC = 2
NS = 16
NW = NC * NS
IMGS_PER_W = BATCH // NW
CH = 16
N_CH = IMGS_PER_W // CH

_mesh = plsc.VectorSubcoreMesh(core_axis_name="c", subcore_axis_name="s")


def _make_sc_kernel(thresh):
    @functools.partial(
        pl.kernel,
        mesh=_mesh,
        compiler_params=pltpu.CompilerParams(needs_layout_passes=False),
        out_type=(
            jax.ShapeDtypeStruct((BATCH, OSTR), jnp.float32),
            jax.ShapeDtypeStruct((NW, IMGS_PER_W), jnp.int32),
        ),
        scratch_types=[
            pltpu.VMEM((CH, NFEAT), jnp.float32),
            pltpu.VMEM((CH, NFEAT), jnp.float32),
            pltpu.VMEM((CH, OSTR), jnp.float32),
            pltpu.VMEM((CH, OSTR), jnp.float32),
            pltpu.VMEM((IMGS_PER_W,), jnp.int32),
            pltpu.SemaphoreType.DMA,
            pltpu.SemaphoreType.DMA,
        ],
    )
    def _sc_map_kernel(src, dst, cnt_hbm, in0, in1, out0, out1, cnt_v,
                       in_sem, out_sem):
        wid = lax.axis_index("s") * NC + lax.axis_index("c")
        lane = lax.iota(jnp.int32, 16)
        inv_s = jnp.float32(1.0 / S)
        one = jnp.ones((16,), jnp.int32)
        zero_i = jnp.zeros((16,), jnp.int32)
        zero_f = jnp.zeros((16,), jnp.float32)

        cgs = []
        for cg in range(3):
            cellv = cg * 16 + lane
            cgs.append((cellv * F, cellv * 6,
                        (cellv % S).astype(jnp.float32),
                        (cellv // S).astype(jnp.float32)))

        def convert(gf, colf, rowf):
            pairs = [(gf(k), jnp.full((16,), jnp.float32(k)))
                     for k in range(C)]
            while len(pairs) > 1:
                nxt = []
                for j in range(0, len(pairs) - 1, 2):
                    (m1, l1), (m2, l2) = pairs[j], pairs[j + 1]
                    right = m2 > m1
                    nxt.append((jnp.where(right, m2, m1),
                                jnp.where(right, l2, l1)))
                if len(pairs) % 2:
                    nxt.append(pairs[-1])
                pairs = nxt
            label = pairs[0][1]

            conf1 = gf(C)
            conf2 = gf(C + 5)
            best = conf2 > conf1
            bb0 = jnp.where(best, gf(C + 6), gf(C + 1))
            bb1 = jnp.where(best, gf(C + 7), gf(C + 2))
            bb2 = jnp.where(best, gf(C + 8), gf(C + 3))
            bb3 = jnp.where(best, gf(C + 9), gf(C + 4))

            cx = (bb0 + colf) * inv_s
            cy = (bb1 + rowf) * inv_s
            w2 = bb2 * inv_s * 0.5
            h2 = bb3 * inv_s * 0.5
            conf = jnp.maximum(conf1, conf2)
            mask = conf > thresh
            outs = (cx - w2, cy - h2, cx + w2, cy + h2, conf, label)
            return mask, [jnp.where(mask, o, zero_f) for o in outs]

        def process_chunk(ci, in_ref, out_ref):
            def img_body(img, cnt_acc):
                imgv = jnp.full((16,), img, jnp.int32)
                msum = zero_i
                for base_in, base_out, colf, rowf in cgs:
                    def gf(f, _b=base_in):
                        return plsc.load_gather(in_ref, [imgv, _b + f])

                    mask, outs = convert(gf, colf, rowf)
                    for k in range(6):
                        plsc.store_scatter(out_ref, [imgv, base_out + k],
                                           outs[k])
                    msum = msum + jnp.where(mask, one, zero_i)
                s = jnp.sum(msum)
                return jnp.where(lane == imgv,
                                 jnp.full((16,), s, jnp.int32), cnt_acc)

            cnt_acc = lax.fori_loop(0, CH, img_body, zero_i)

            c48 = (CELLS - 1) * F

            def gf48(f):
                return plsc.load_gather(
                    in_ref, [lane, jnp.full((16,), c48 + f, jnp.int32)])

            col48 = jnp.full((16,), jnp.float32((CELLS - 1) % S))
            row48 = jnp.full((16,), jnp.float32((CELLS - 1) // S))
            mask48, outs48 = convert(gf48, col48, row48)
            ob48 = (CELLS - 1) * 6
            for k in range(6):
                plsc.store_scatter(
                    out_ref, [lane, jnp.full((16,), ob48 + k, jnp.int32)],
                    outs48[k])
            cnt_acc = cnt_acc + jnp.where(mask48, one, zero_i)
            cnt_v[pl.ds(ci * CH, CH)] = cnt_acc

        base_img = wid * IMGS_PER_W
        bufs = ((in0, out0), (in1, out1))

        def start_in(ci, b):
            pltpu.async_copy(src.at[pl.ds(base_img + ci * CH, CH), :],
                             bufs[b][0], in_sem)

        def wait_in(b):
            pltpu.make_async_copy(src.at[pl.ds(0, CH), :],
                                  bufs[b][0], in_sem).wait()

        def start_out(ci, b):
            pltpu.async_copy(bufs[b][1],
                             dst.at[pl.ds(base_img + ci * CH, CH), :],
                             out_sem)

        def wait_out(b):
            pltpu.make_async_copy(bufs[b][1], dst.at[pl.ds(0, CH), :],
                                  out_sem).wait()

        def half(ci, b, first_pair):
            wait_in(b)

            @pl.when(jnp.logical_not(first_pair))
            def _():
                wait_out(b)

            process_chunk(ci, bufs[b][0], bufs[b][1])
            start_out(ci, b)

            @pl.when(ci + 2 < N_CH)
            def _():
                start_in(ci + 2, b)

        start_in(0, 0)
        start_in(1, 1)

        def pair_body(cp, _):
            ci0 = cp * 2
            first = cp == 0
            half(ci0, 0, first)
            half(ci0 + 1, 1, first)
            return 0

        lax.fori_loop(0, N_CH // 2, pair_body, 0)
        wait_out(0)
        wait_out(1)
        pltpu.sync_copy(cnt_v, cnt_hbm.at[wid])

    return _sc_map_kernel


_pred_kernel = _make_sc_kernel(jnp.float32(0.1))
_tgt_kernel = _make_sc_kernel(jnp.float32(0.5))


def kernel(predictions, targets):
    mp, pc = _pred_kernel(predictions)
    mt, tc = _tgt_kernel(targets)
    return (mp.reshape(BATCH, PCELLS, 6)[:, :CELLS, :],
            mt.reshape(BATCH, PCELLS, 6)[:, :CELLS, :],
            pc.reshape(BATCH),
            tc.reshape(BATCH))
